# baseline (speedup 1.0000x reference)
import jax
import jax.numpy as jnp
from jax import lax
from jax.experimental import pallas as pl
from jax.experimental.pallas import tpu as pltpu
from jax.experimental.pallas import tpu_sc as plsc

NSEL = 100
NKP = 17
KP = NKP * 3
KP_PAD = 52
SC_PAD = 112
L = 16
NBINS = 8192
SHIFT = 19
TOPBIN = 2048
CAP = 1024
GW = 128
BXCH = NSEL * 4 // GW + 1
KPCH = (NSEL * KP + GW - 1) // GW


def _pp_body(prob_hbm, boxes_hbm, kpts_hbm, swh_hbm,
             scores_hbm, boxes_out_hbm, kpts_out_hbm,
             data_v, hist_v, cand_k_v, cand_i_v, swh_v, scores_v, qsel_v,
             bidx_v, kidx_v, bx_raw_v, kp_raw_v, bx_out_v, kp_out_v,
             sem_b, sem_k):
    cid = lax.axis_index("c")
    sid = lax.axis_index("s")
    wid = sid * 2 + cid
    nb = prob_hbm.shape[0]
    nflat = prob_hbm.shape[1]
    nq = boxes_hbm.shape[0] // (nb * 4)
    nbx = boxes_hbm.shape[0]
    nkp = kpts_hbm.shape[0]

    @pl.when(wid < nb)
    def _():
        b = wid
        pltpu.sync_copy(prob_hbm.at[b], data_v)
        pltpu.sync_copy(swh_hbm.at[b], swh_v)

        iota = lax.iota(jnp.int32, L)
        zeros_i = jnp.zeros((L,), jnp.int32)
        ones_i = jnp.ones((L,), jnp.int32)

        def clr(g, _):
            hist_v[pl.ds(g * L, L)] = zeros_i
            return 0

        lax.fori_loop(0, NBINS // L, clr, 0)

        def histo(g, _):
            k = lax.bitcast_convert_type(data_v[pl.ds(g * L, L)], jnp.int32)
            plsc.addupdate_scatter(hist_v, [k >> SHIFT], ones_i)
            return 0

        lax.fori_loop(0, nflat // L, histo, 0)

        def scan_step(j, carry):
            acc, found, bbin = carry
            base = TOPBIN - (j + 1) * L
            h = hist_v[pl.ds(base, L)]
            rev = lax.rev(h, (0,))
            cum = plsc.cumsum(rev) + acc
            hit = cum >= NSEL
            npop = jnp.max(plsc.all_reduce_population_count(hit))
            b_here = base + npop - 1
            upd = (found == 0) & (npop > 0)
            bbin = jnp.where(upd, b_here, bbin)
            found = jnp.where(npop > 0, 1, found)
            return acc + jnp.sum(h), found, bbin

        _, _, bbin = lax.fori_loop(0, TOPBIN // L, scan_step, (0, 0, 0))
        thresh = bbin << SHIFT

        def coll(g, cnt):
            k = lax.bitcast_convert_type(data_v[pl.ds(g * L, L)], jnp.int32)
            msk = k >= thresh
            pcs = plsc.cumsum(jnp.where(msk, 1, 0))
            pos = cnt + pcs - 1
            okm = msk & (pos < CAP)
            plsc.store_scatter(cand_k_v, [pos], k, mask=okm)
            plsc.store_scatter(cand_i_v, [pos], g * L + iota, mask=okm)
            return cnt + jnp.max(pcs)

        cnt = lax.fori_loop(0, nflat // L, coll, 0)
        ncand = jnp.minimum(cnt, CAP)

        def zsc(g, _):
            scores_v[pl.ds(g * L, L)] = jnp.zeros((L,), jnp.float32)
            return 0

        lax.fori_loop(0, SC_PAD // L, zsc, 0)

        def rank_chunk(t, _):
            post = t * L + iota
            kt = cand_k_v[pl.ds(t * L, L)]

            def inner(j, r):
                kj = cand_k_v[pl.ds(j, L)][0]
                return (r + jnp.where(kj > kt, 1, 0)
                        + jnp.where((kj == kt) & (j < post), 1, 0))

            rank = lax.fori_loop(0, ncand, inner, zeros_i)
            msk = (post < ncand) & (rank < NSEL)
            plsc.store_scatter(scores_v, [rank],
                               lax.bitcast_convert_type(kt, jnp.float32),
                               mask=msk)
            qi = cand_i_v[pl.ds(t * L, L)]
            plsc.store_scatter(qsel_v, [rank], b * nq + (qi >> 1), mask=msk)
            return 0

        lax.fori_loop(0, (ncand + L - 1) // L, rank_chunk, 0)

        def mk_bidx(g, _):
            f = g * L + iota
            r = jnp.minimum(f >> 2, NSEL - 1)
            cc = f & 3
            qv = plsc.load_gather(qsel_v, [r])
            plsc.store_scatter(bidx_v, [f >> 7, f & (GW - 1)],
                               jnp.minimum(qv * 4 + cc, nbx - 1))
            return 0

        lax.fori_loop(0, BXCH * GW // L, mk_bidx, 0)

        def mk_kidx(g, _):
            f = g * L + iota
            r = jnp.minimum(f // KP, NSEL - 1)
            cc = jnp.minimum(f - r * KP, KP - 1)
            qv = plsc.load_gather(qsel_v, [r])
            plsc.store_scatter(kidx_v, [f >> 7, f & (GW - 1)],
                               jnp.minimum(qv * KP + cc, nkp - 1))
            return 0

        lax.fori_loop(0, KPCH * GW // L, mk_kidx, 0)

        def fire_b(j, _):
            pltpu.async_copy(boxes_hbm.at[bidx_v.at[j]], bx_raw_v.at[j], sem_b)
            return 0

        lax.fori_loop(0, BXCH, fire_b, 0)

        def fire_k(j, _):
            pltpu.async_copy(kpts_hbm.at[kidx_v.at[j]], kp_raw_v.at[j], sem_k)
            return 0

        lax.fori_loop(0, KPCH, fire_k, 0)

        def drain_b(j, _):
            pltpu.make_async_copy(boxes_hbm.at[bidx_v.at[j]],
                                  bx_raw_v.at[j], sem_b).wait()
            return 0

        lax.fori_loop(0, BXCH, drain_b, 0)

        def drain_k(j, _):
            pltpu.make_async_copy(kpts_hbm.at[kidx_v.at[j]],
                                  kp_raw_v.at[j], sem_k).wait()
            return 0

        lax.fori_loop(0, KPCH, drain_k, 0)

        svec = swh_v[...]

        def bx(g, _):
            o = g * L + iota
            r = o >> 2
            cc = o & 3
            fa = r * 4 + (o & 1)
            a = plsc.load_gather(bx_raw_v, [fa >> 7, fa & (GW - 1)])
            fw = fa + 2
            wd = plsc.load_gather(bx_raw_v, [fw >> 7, fw & (GW - 1)])
            sgn = jnp.where(cc < 2, -0.5, 0.5)
            plsc.store_scatter(bx_out_v, [r, cc], (a + sgn * wd) * svec)
            return 0

        lax.fori_loop(0, NSEL * 4 // L, bx, 0)

        w_s = svec[0]
        h_s = svec[1]

        def kp(g, _):
            o = g * L + iota
            r = o // KP_PAD
            cc = o - r * KP_PAD
            c3 = cc % 3
            cd3 = cc // 3
            j = jnp.where(c3 == 0, 2 * cd3,
                          jnp.where(c3 == 1, 2 * cd3 + 1, 34 + cd3))
            f = r * KP + jnp.minimum(j, KP - 1)
            val = plsc.load_gather(kp_raw_v, [f >> 7, f & (GW - 1)])
            scv = jnp.where(c3 == 0, w_s, jnp.where(c3 == 1, h_s, 1.0))
            scv = jnp.where(cc == KP, 0.0, scv)
            plsc.store_scatter(kp_out_v, [r, cc], val * scv)
            return 0

        lax.fori_loop(0, NSEL * KP_PAD // L, kp, 0)

        pltpu.sync_copy(scores_v, scores_hbm.at[b])
        pltpu.sync_copy(bx_out_v, boxes_out_hbm.at[b])
        pltpu.sync_copy(kp_out_v, kpts_out_hbm.at[b])


def _pp_call(prob, boxes_flat, kpts_flat, swh):
    bs, nflat = prob.shape
    mesh = plsc.VectorSubcoreMesh(core_axis_name="c", subcore_axis_name="s")
    fn = pl.kernel(
        _pp_body,
        out_type=(
            jax.ShapeDtypeStruct((bs, SC_PAD), jnp.float32),
            jax.ShapeDtypeStruct((bs, NSEL, 4), jnp.float32),
            jax.ShapeDtypeStruct((bs, NSEL, KP_PAD), jnp.float32),
        ),
        mesh=mesh,
        compiler_params=pltpu.CompilerParams(
            needs_layout_passes=False, use_tc_tiling_on_sc=False),
        scratch_types=[
            pltpu.VMEM((nflat,), jnp.float32),
            pltpu.VMEM((NBINS,), jnp.int32),
            pltpu.VMEM((CAP + L,), jnp.int32),
            pltpu.VMEM((CAP + L,), jnp.int32),
            pltpu.VMEM((L,), jnp.float32),
            pltpu.VMEM((SC_PAD,), jnp.float32),
            pltpu.VMEM((NSEL,), jnp.int32),
            pltpu.VMEM((BXCH, GW), jnp.int32),
            pltpu.VMEM((KPCH, GW), jnp.int32),
            pltpu.VMEM((BXCH, GW), jnp.float32),
            pltpu.VMEM((KPCH, GW), jnp.float32),
            pltpu.VMEM((NSEL, 4), jnp.float32),
            pltpu.VMEM((NSEL, KP_PAD), jnp.float32),
            pltpu.SemaphoreType.DMA,
            pltpu.SemaphoreType.DMA,
        ],
    )
    return fn(prob, boxes_flat, kpts_flat, swh)


def kernel(pred_logits, pred_boxes, pred_keypoints, orig_target_sizes, target_sizes):
    bs, nq, nc = pred_logits.shape
    prob = jax.nn.sigmoid(pred_logits).reshape(bs, nq * nc)
    boxes_flat = pred_boxes.reshape(bs * nq * 4)
    kpts_flat = pred_keypoints.reshape(bs * nq * KP)
    whf = orig_target_sizes.astype(jnp.float32)
    swh = jnp.tile(jnp.stack([whf[:, 1], whf[:, 0]], axis=1), (1, L // 2))

    scores_p, boxes, kpts_p = _pp_call(prob, boxes_flat, kpts_flat, swh)

    scores = scores_p[:, :NSEL]
    kpts = kpts_p[:, :, :KP]
    labels = jnp.ones((bs, NSEL), jnp.int32)
    return scores, labels, boxes, kpts, kpts[:, 0]


# speedup vs baseline: 1.6974x; 1.6974x over previous
import jax
import jax.numpy as jnp
from jax import lax
from jax.experimental import pallas as pl
from jax.experimental.pallas import tpu as pltpu
from jax.experimental.pallas import tpu_sc as plsc

NSEL = 100
NKP = 17
KP = NKP * 3
KP_PAD = 52
SC_PAD = 112
L = 16
NBINS = 8192
SHIFT = 19
TOPBIN = 2048
CAP = 1024
GW = 128
BXCH = NSEL * 4 // GW + 1
KPCH = (NSEL * KP + GW - 1) // GW


def _pp_body(prob_hbm, boxes_hbm, kpts_hbm, swh_hbm,
             scores_hbm, boxes_out_hbm, kpts_out_hbm,
             data_v, hist_v, cand_k_v, cand_i_v, swh_v, scores_v, qsel_v,
             bx_raw_v, kp_raw_v, bx_out_v, kp_out_v, sem_b, sem_k):
    cid = lax.axis_index("c")
    sid = lax.axis_index("s")
    wid = sid * 2 + cid
    nb = prob_hbm.shape[0]
    nflat = prob_hbm.shape[1]
    nq = boxes_hbm.shape[1]

    @pl.when(wid < nb)
    def _():
        b = wid
        pltpu.sync_copy(prob_hbm.at[b], data_v)
        pltpu.sync_copy(swh_hbm.at[b], swh_v)

        iota = lax.iota(jnp.int32, L)
        zeros_i = jnp.zeros((L,), jnp.int32)
        ones_i = jnp.ones((L,), jnp.int32)

        def clr(g, _):
            hist_v[pl.ds(g * L, L)] = zeros_i
            return 0

        lax.fori_loop(0, NBINS // L, clr, 0)

        def histo(g, _):
            k = lax.bitcast_convert_type(data_v[pl.ds(g * L, L)], jnp.int32)
            plsc.addupdate_scatter(hist_v, [k >> SHIFT], ones_i)
            return 0

        lax.fori_loop(0, nflat // L, histo, 0)

        def scan_step(j, carry):
            acc, found, bbin = carry
            base = TOPBIN - (j + 1) * L
            h = hist_v[pl.ds(base, L)]
            rev = lax.rev(h, (0,))
            cum = plsc.cumsum(rev) + acc
            hit = cum >= NSEL
            npop = jnp.max(plsc.all_reduce_population_count(hit))
            b_here = base + npop - 1
            upd = (found == 0) & (npop > 0)
            bbin = jnp.where(upd, b_here, bbin)
            found = jnp.where(npop > 0, 1, found)
            return acc + jnp.sum(h), found, bbin

        _, _, bbin = lax.fori_loop(0, TOPBIN // L, scan_step, (0, 0, 0))
        thresh = bbin << SHIFT

        def coll(g, cnt):
            k = lax.bitcast_convert_type(data_v[pl.ds(g * L, L)], jnp.int32)
            msk = k >= thresh
            pcs = plsc.cumsum(jnp.where(msk, 1, 0))
            pos = cnt + pcs - 1
            okm = msk & (pos < CAP)
            plsc.store_scatter(cand_k_v, [pos], k, mask=okm)
            plsc.store_scatter(cand_i_v, [pos], g * L + iota, mask=okm)
            return cnt + jnp.max(pcs)

        cnt = lax.fori_loop(0, nflat // L, coll, 0)
        ncand = jnp.minimum(cnt, CAP)

        def zsc(g, _):
            scores_v[pl.ds(g * L, L)] = jnp.zeros((L,), jnp.float32)
            return 0

        lax.fori_loop(0, SC_PAD // L, zsc, 0)

        def rank_chunk(t, _):
            post = t * L + iota
            kt = cand_k_v[pl.ds(t * L, L)]

            def inner(j, r):
                kj = cand_k_v[pl.ds(j, L)][0]
                return (r + jnp.where(kj > kt, 1, 0)
                        + jnp.where((kj == kt) & (j < post), 1, 0))

            rank = lax.fori_loop(0, ncand, inner, zeros_i)
            msk = (post < ncand) & (rank < NSEL)
            plsc.store_scatter(scores_v, [rank],
                               lax.bitcast_convert_type(kt, jnp.float32),
                               mask=msk)
            qi = cand_i_v[pl.ds(t * L, L)]
            plsc.store_scatter(qsel_v, [rank], qi >> 1, mask=msk)
            return 0

        lax.fori_loop(0, (ncand + L - 1) // L, rank_chunk, 0)

        def fire(r, _):
            q = qsel_v[pl.ds(r, L)][0]
            pltpu.async_copy(boxes_hbm.at[b, q], bx_raw_v.at[r, pl.ds(0, 4)],
                             sem_b)
            pltpu.async_copy(kpts_hbm.at[b, q], kp_raw_v.at[r, pl.ds(0, KP)],
                             sem_k)
            return 0

        lax.fori_loop(0, NSEL, fire, 0)

        def drain(r, _):
            q = qsel_v[pl.ds(r, L)][0]
            pltpu.make_async_copy(boxes_hbm.at[b, q],
                                  bx_raw_v.at[r, pl.ds(0, 4)], sem_b).wait()
            pltpu.make_async_copy(kpts_hbm.at[b, q],
                                  kp_raw_v.at[r, pl.ds(0, KP)], sem_k).wait()
            return 0

        lax.fori_loop(0, NSEL, drain, 0)

        svec = swh_v[...]

        def bx(g, _):
            o = g * L + iota
            r = o >> 2
            cc = o & 3
            p = o & 1
            a = plsc.load_gather(bx_raw_v, [r, p])
            wd = plsc.load_gather(bx_raw_v, [r, p + 2])
            sgn = jnp.where(cc < 2, -0.5, 0.5)
            plsc.store_scatter(bx_out_v, [r, cc], (a + sgn * wd) * svec)
            return 0

        lax.fori_loop(0, NSEL * 4 // L, bx, 0)

        w_s = svec[0]
        h_s = svec[1]

        def kp(g, _):
            o = g * L + iota
            r = o // KP_PAD
            cc = o - r * KP_PAD
            c3 = cc % 3
            cd3 = cc // 3
            j = jnp.where(c3 == 0, 2 * cd3,
                          jnp.where(c3 == 1, 2 * cd3 + 1, 34 + cd3))
            val = plsc.load_gather(kp_raw_v, [r, j])
            scv = jnp.where(c3 == 0, w_s, jnp.where(c3 == 1, h_s, 1.0))
            scv = jnp.where(cc == KP, 0.0, scv)
            plsc.store_scatter(kp_out_v, [r, cc], val * scv)
            return 0

        lax.fori_loop(0, NSEL * KP_PAD // L, kp, 0)

        pltpu.sync_copy(scores_v, scores_hbm.at[b])
        pltpu.sync_copy(bx_out_v, boxes_out_hbm.at[b])
        pltpu.sync_copy(kp_out_v, kpts_out_hbm.at[b])


def _pp_call(prob, boxes_flat, kpts_flat, swh):
    bs, nflat = prob.shape
    mesh = plsc.VectorSubcoreMesh(core_axis_name="c", subcore_axis_name="s")
    fn = pl.kernel(
        _pp_body,
        out_type=(
            jax.ShapeDtypeStruct((bs, SC_PAD), jnp.float32),
            jax.ShapeDtypeStruct((bs, NSEL, 4), jnp.float32),
            jax.ShapeDtypeStruct((bs, NSEL, KP_PAD), jnp.float32),
        ),
        mesh=mesh,
        compiler_params=pltpu.CompilerParams(
            needs_layout_passes=False, use_tc_tiling_on_sc=True),
        scratch_types=[
            pltpu.VMEM((nflat,), jnp.float32),
            pltpu.VMEM((NBINS,), jnp.int32),
            pltpu.VMEM((CAP + L,), jnp.int32),
            pltpu.VMEM((CAP + L,), jnp.int32),
            pltpu.VMEM((L,), jnp.float32),
            pltpu.VMEM((SC_PAD,), jnp.float32),
            pltpu.VMEM((NSEL + L,), jnp.int32),
            pltpu.VMEM((NSEL, 8), jnp.float32),
            pltpu.VMEM((NSEL, 56), jnp.float32),
            pltpu.VMEM((NSEL, 4), jnp.float32),
            pltpu.VMEM((NSEL, KP_PAD), jnp.float32),
            pltpu.SemaphoreType.DMA,
            pltpu.SemaphoreType.DMA,
        ],
    )
    return fn(prob, boxes_flat, kpts_flat, swh)


def kernel(pred_logits, pred_boxes, pred_keypoints, orig_target_sizes, target_sizes):
    bs, nq, nc = pred_logits.shape
    prob = jax.nn.sigmoid(pred_logits).reshape(bs, nq * nc)
    boxes_flat = pred_boxes
    kpts_flat = pred_keypoints
    whf = orig_target_sizes.astype(jnp.float32)
    swh = jnp.tile(jnp.stack([whf[:, 1], whf[:, 0]], axis=1), (1, L // 2))

    scores_p, boxes, kpts_p = _pp_call(prob, boxes_flat, kpts_flat, swh)

    scores = scores_p[:, :NSEL]
    kpts = kpts_p[:, :, :KP]
    labels = jnp.ones((bs, NSEL), jnp.int32)
    return scores, labels, boxes, kpts, kpts[:, 0]
